# baseline (speedup 1.0000x reference)
import functools

import jax
import jax.numpy as jnp
from jax import lax
from jax.experimental import pallas as pl
from jax.experimental.pallas import tpu as pltpu
from jax.experimental.pallas import tpu_sc as plsc

N = 10000
NPAD = 10112
E = 320000
D = 128
NG = 64
NTILES = 32
EPT = E // NTILES
C = 128
NCH = 80
SEG = NCH // 2
EPTP = NCH * C
RPT = NPAD // 16
HW = 8

_mesh = functools.partial(
    plsc.VectorSubcoreMesh, core_axis_name="c", subcore_axis_name="s")


def _zero_acc_slice(zeros_hbm, acc_sh, base):
    for k in range(RPT // 128):
        pltpu.sync_copy(zeros_hbm, acc_sh.at[pl.ds(base + k * 128, 128)])
    rem = RPT % 128
    if rem:
        pltpu.sync_copy(zeros_hbm.at[pl.ds(0, rem)],
                        acc_sh.at[pl.ds(base + (RPT // 128) * 128, rem)])


def _hist_body(dst_hbm, ones_hbm, zeros_hbm, out_hbm, idx_v, ones_v, acc_sh, sem):
    c = lax.axis_index("c")
    s = lax.axis_index("s")
    wid = c * 16 + s
    base = s * RPT
    _zero_acc_slice(zeros_hbm, acc_sh, base)
    pltpu.sync_copy(ones_hbm, ones_v)
    pltpu.sync_copy(dst_hbm.at[wid], idx_v)
    plsc.subcore_barrier()

    def step(j, carry):
        pltpu.sync_copy(ones_v, acc_sh.at[idx_v.at[j]], add=True)
        return carry

    lax.fori_loop(0, NCH, step, 0)
    plsc.subcore_barrier()
    pltpu.sync_copy(acc_sh.at[pl.ds(base, RPT)],
                    out_hbm.at[c].at[pl.ds(base, RPT)])


@functools.partial(
    pl.kernel,
    mesh=_mesh(),
    out_type=jax.ShapeDtypeStruct((2, NPAD, HW), jnp.float32),
    scratch_types=[
        pltpu.VMEM((NCH, C), jnp.int32),
        pltpu.VMEM((128, HW), jnp.float32),
        pltpu.VMEM_SHARED((NPAD, HW), jnp.float32),
        pltpu.SemaphoreType.DMA,
    ],
)
def _sc_hist(dst_hbm, ones_hbm, zeros_hbm, out_hbm, idx_v, ones_v, acc_sh, sem):
    _hist_body(dst_hbm, ones_hbm, zeros_hbm, out_hbm, idx_v, ones_v, acc_sh, sem)


def _mid_body(x_ref, W_ref, hist_ref, g_ref, dinv_ref):
    deg = hist_ref[0, :, :1] + hist_ref[1, :, :1] + 1.0
    dinv = jax.lax.rsqrt(deg)
    dinv_ref[...] = dinv
    h = jax.lax.dot_general(x_ref[...], W_ref[...], (((1,), (0,)), ((), ())),
                            preferred_element_type=jnp.float32)
    g_ref[:N, :] = h * dinv[:N]
    g_ref[N:, :] = jnp.zeros((NPAD - N, D), jnp.float32)


def _tc_mid(x, W_conv, hist):
    return pl.pallas_call(
        _mid_body,
        out_shape=(jax.ShapeDtypeStruct((NPAD, D), jnp.float32),
                   jax.ShapeDtypeStruct((NPAD, 1), jnp.float32)),
    )(x, W_conv, hist)


def _scat_body(g_hbm, src_hbm, dst_hbm, zeros_hbm, out_hbm,
               src_v, dst_v, buf0, buf1, acc_sh, sem0, sem1):
    c = lax.axis_index("c")
    s = lax.axis_index("s")
    wid = c * 16 + s
    base = s * RPT
    _zero_acc_slice(zeros_hbm, acc_sh, base)
    plsc.subcore_barrier()

    for seg_start in (0, SEG):
        pltpu.sync_copy(src_hbm.at[wid].at[pl.ds(seg_start, SEG)], src_v)
        pltpu.sync_copy(dst_hbm.at[wid].at[pl.ds(seg_start, SEG)], dst_v)
        pltpu.async_copy(g_hbm.at[src_v.at[0]], buf0, sem0)

        def step(i, carry):
            j = 2 * i
            jp = jnp.where(j == 0, 0, j)
            pltpu.make_async_copy(g_hbm.at[src_v.at[jp]], buf0, sem0).wait()
            pltpu.async_copy(g_hbm.at[src_v.at[j + 1]], buf1, sem1)
            pltpu.sync_copy(buf0, acc_sh.at[dst_v.at[j]], add=True)
            jn = jnp.where(j + 2 < SEG, j + 2, 0)
            pltpu.make_async_copy(g_hbm.at[src_v.at[j + 1]], buf1, sem1).wait()
            pltpu.async_copy(g_hbm.at[src_v.at[jn]], buf0, sem0)
            pltpu.sync_copy(buf1, acc_sh.at[dst_v.at[j + 1]], add=True)
            return carry

        lax.fori_loop(0, SEG // 2, step, 0)
        pltpu.make_async_copy(g_hbm.at[src_v.at[0]], buf0, sem0).wait()
    plsc.subcore_barrier()
    pltpu.sync_copy(acc_sh.at[pl.ds(base, RPT)],
                    out_hbm.at[c].at[pl.ds(base, RPT)])


@functools.partial(
    pl.kernel,
    mesh=_mesh(),
    out_type=jax.ShapeDtypeStruct((2, NPAD, D), jnp.float32),
    scratch_types=[
        pltpu.VMEM((SEG, C), jnp.int32),
        pltpu.VMEM((SEG, C), jnp.int32),
        pltpu.VMEM((C, D), jnp.float32),
        pltpu.VMEM((C, D), jnp.float32),
        pltpu.VMEM_SHARED((NPAD, D), jnp.float32),
        pltpu.SemaphoreType.DMA,
        pltpu.SemaphoreType.DMA,
    ],
)
def _sc_scatter(g_hbm, src_hbm, dst_hbm, zeros_hbm, out_hbm,
                src_v, dst_v, buf0, buf1, acc_sh, sem0, sem1):
    _scat_body(g_hbm, src_hbm, dst_hbm, zeros_hbm, out_hbm,
               src_v, dst_v, buf0, buf1, acc_sh, sem0, sem1)


def _finish_body(parts_ref, g_ref, dinv_ref, batch_ref, b_conv_ref,
                 W1_ref, b1_ref, W2_ref, b2_ref, out_ref):
    acc = parts_ref[0, :N, :] + parts_ref[1, :N, :] + g_ref[:N, :]
    out_node = jnp.maximum(acc * dinv_ref[:N] + b_conv_ref[...], 0.0)
    gids = jax.lax.broadcasted_iota(jnp.int32, (N, NG), 1)
    P = jnp.where(batch_ref[...] == gids, 1.0, 0.0)
    sums = jax.lax.dot_general(P, out_node, (((0,), (0,)), ((), ())),
                               preferred_element_type=jnp.float32)
    cnts = jnp.sum(P, axis=0, keepdims=True)
    pooled = sums / jnp.maximum(cnts, 1.0).T
    z = jnp.maximum(
        jax.lax.dot_general(pooled, W1_ref[...], (((1,), (0,)), ((), ())),
                            preferred_element_type=jnp.float32) + b1_ref[...], 0.0)
    out_ref[...] = jax.lax.dot_general(z, W2_ref[...], (((1,), (0,)), ((), ())),
                                       preferred_element_type=jnp.float32) + b2_ref[...]


def _finish(parts, g, dinv, batch, b_conv, W1, b1, W2, b2):
    return pl.pallas_call(
        _finish_body,
        out_shape=jax.ShapeDtypeStruct((NG, 10), jnp.float32),
    )(parts, g, dinv, batch.reshape(N, 1), b_conv.reshape(1, -1),
      W1, b1.reshape(1, -1), W2, b2.reshape(1, -1))


def kernel(x, edge_index, batch, W_conv, b_conv, W1, b1, W2, b2):
    src = edge_index[0]
    dst = edge_index[1]
    pad = jnp.full((NTILES, EPTP - EPT), N, jnp.int32)
    src3 = jnp.concatenate([src.reshape(NTILES, EPT), pad], 1).reshape(
        NTILES, NCH, C)
    dst3 = jnp.concatenate([dst.reshape(NTILES, EPT), pad], 1).reshape(
        NTILES, NCH, C)
    ones_h = jnp.ones((128, HW), jnp.float32)
    zeros_h = jnp.zeros((128, HW), jnp.float32)
    hist = _sc_hist(dst3, ones_h, zeros_h)
    g_pad, dinv = _tc_mid(x, W_conv, hist)
    zeros2d = jnp.zeros((128, D), jnp.float32)
    parts = _sc_scatter(g_pad, src3, dst3, zeros2d)
    return _finish(parts, g_pad, dinv, batch, b_conv, W1, b1, W2, b2)
